# Initial kernel scaffold; baseline (speedup 1.0000x reference)
import jax
import jax.numpy as jnp
from jax.experimental import pallas as pl


def kernel(x, pair_roles, W_sub, b_sub, W_obj, b_obj, W_val, b_val):
    raise NotImplementedError("write your pallas kernel here")


# speedup vs baseline: 1.1673x; 1.1673x over previous
import jax
import jax.numpy as jnp
from jax.experimental import pallas as pl
from jax.experimental.pallas import tpu as pltpu

HIDDEN = 2048
HEADS = 16
T = 512
OT = 512


def _body(roles_ref, x_ref, ws_ref, wo_ref, wv_ref, bs_ref, bo_ref, bv_ref, out_ref):
    x = x_ref[...]
    dims = (((1,), (1,)), ((), ()))
    sub = jax.lax.dot_general(x, ws_ref[...], dims, preferred_element_type=jnp.float32) + bs_ref[...]
    obj = jax.lax.dot_general(x, wo_ref[...], dims, preferred_element_type=jnp.float32) + bo_ref[...]
    val = jax.lax.dot_general(x, wv_ref[...], dims, preferred_element_type=jnp.float32) + bv_ref[...]
    role = roles_ref[...]
    out_ref[...] = jnp.where(role == 0, sub, jnp.where(role == 1, obj, val))


def kernel(x, pair_roles, W_sub, b_sub, W_obj, b_obj, W_val, b_val):
    B, P, H = x.shape
    N = B * P
    xf = x.reshape(N, H)
    roles = jnp.broadcast_to(pair_roles.astype(jnp.int32)[None, :], (B, P)).reshape(N, 1)

    grid = (H // OT, N // T)

    out = pl.pallas_call(
        _body,
        grid=grid,
        in_specs=[
            pl.BlockSpec((T, 1), lambda o, t: (t, 0)),
            pl.BlockSpec((T, H), lambda o, t: (t, 0)),
            pl.BlockSpec((OT, H), lambda o, t: (o, 0)),
            pl.BlockSpec((OT, H), lambda o, t: (o, 0)),
            pl.BlockSpec((OT, H), lambda o, t: (o, 0)),
            pl.BlockSpec((1, OT), lambda o, t: (0, o)),
            pl.BlockSpec((1, OT), lambda o, t: (0, o)),
            pl.BlockSpec((1, OT), lambda o, t: (0, o)),
        ],
        out_specs=pl.BlockSpec((T, OT), lambda o, t: (t, o)),
        out_shape=jax.ShapeDtypeStruct((N, H), jnp.float32),
    )(roles, xf, W_sub, W_obj, W_val,
      b_sub.reshape(1, H), b_obj.reshape(1, H), b_val.reshape(1, H))

    return out.reshape(B, P, HEADS, H // HEADS)
